# baseline (speedup 1.0000x reference)
import functools

import jax
import jax.numpy as jnp
from jax import lax
from jax.experimental import pallas as pl
from jax.experimental.pallas import tpu as pltpu
from jax.experimental.pallas import tpu_sc as plsc

_M = 256


def _silu(u):
    return u / (1.0 + jnp.exp(-u))


def _sigmoid(u):
    return 1.0 / (1.0 + jnp.exp(-u))


def _dot_nt(a, b):
    return lax.dot_general(a, b, (((1,), (1,)), ((), ())),
                           preferred_element_type=jnp.float32)


def _dot_nn(a, b):
    return lax.dot_general(a, b, (((1,), (0,)), ((), ())),
                           preferred_element_type=jnp.float32)


def _router_body(x_ref, gw_ref, d0_ref, d1_ref, w0_ref, w1_ref, be_ref,
                 *, m_blk, nb):
    logits = _dot_nt(x_ref[...], gw_ref[...])
    T, E = logits.shape
    lane = lax.broadcasted_iota(jnp.int32, (T, E), 1)
    m1 = jnp.max(logits, axis=1, keepdims=True)
    i1 = jnp.min(jnp.where(logits == m1, lane, E), axis=1, keepdims=True)
    masked = jnp.where(lane == i1, -jnp.inf, logits)
    m2 = jnp.max(masked, axis=1, keepdims=True)
    i2 = jnp.min(jnp.where(masked == m2, lane, E), axis=1, keepdims=True)
    WL = w0_ref.shape[1]
    wa = 1.0 / (1.0 + jnp.exp(m2 - m1))
    zeros_wl = jnp.zeros((T, WL), jnp.float32)
    w0_ref[...] = wa + zeros_wl
    w1_ref[...] = (1.0 - wa) + zeros_wl

    O1 = (lane == i1).astype(jnp.float32)
    O2 = (lane == i2).astype(jnp.float32)

    C = 512
    r_io = lax.broadcasted_iota(jnp.int32, (C, C), 0)
    c_io = lax.broadcasted_iota(jnp.int32, (C, C), 1)
    LT = (r_io >= c_io).astype(jnp.float32)
    n_chunk_half = T // C
    counts = jnp.zeros((1, E), jnp.float32)
    ranks = []
    for c in range(2 * n_chunk_half):
        if c < n_chunk_half:
            Oc = O1[c * C:(c + 1) * C, :]
        else:
            Oc = O2[(c - n_chunk_half) * C:(c - n_chunk_half + 1) * C, :]
        pref = _dot_nn(LT, Oc)
        ranks.append(jnp.sum((pref + counts) * Oc, axis=1, keepdims=True)
                     - 1.0)
        counts = counts + jnp.sum(Oc, axis=0, keepdims=True)

    cnt_i = counts.astype(jnp.int32)
    pc = ((cnt_i + m_blk - 1) // m_blk) * m_blk
    pc_f = pc.astype(jnp.float32)
    re_io = lax.broadcasted_iota(jnp.int32, (E, E), 0)
    ce_io = lax.broadcasted_iota(jnp.int32, (E, E), 1)
    U = (re_io < ce_io).astype(jnp.float32)
    off = _dot_nn(pc_f, U)

    off1 = jnp.sum(off * O1, axis=1, keepdims=True)
    off2 = jnp.sum(off * O2, axis=1, keepdims=True)
    rank = jnp.concatenate(ranks, axis=0)
    d0_ref[...] = (off1 + rank[:T, :]).astype(jnp.int32)
    d1_ref[...] = (off2 + rank[T:, :]).astype(jnp.int32)

    off_end = off + pc_f
    eyeE = (re_io == ce_io).astype(jnp.float32)
    oe_col = _dot_nt(eyeE, off_end)
    brow = (lax.broadcasted_iota(jnp.int32, (1, nb), 1) * m_blk
            ).astype(jnp.float32)
    cmp = (brow >= oe_col).astype(jnp.int32)
    be_ref[...] = jnp.minimum(jnp.sum(cmp, axis=0, keepdims=True), E - 1)


def _router(x, gate_w, m_blk, nb):
    T, D = x.shape
    E = gate_w.shape[0]
    body = functools.partial(_router_body, m_blk=m_blk, nb=nb)
    return pl.pallas_call(
        body,
        out_shape=[
            jax.ShapeDtypeStruct((T, 1), jnp.int32),
            jax.ShapeDtypeStruct((T, 1), jnp.int32),
            jax.ShapeDtypeStruct((T, 128), jnp.float32),
            jax.ShapeDtypeStruct((T, 128), jnp.float32),
            jax.ShapeDtypeStruct((1, nb), jnp.int32),
        ],
    )(x, gate_w)


def _sc_dispatch(x, d0, d1, w0, w1, s_rows):
    T, D = x.shape
    NW = 32
    TPW = T // NW
    mesh = plsc.VectorSubcoreMesh(core_axis_name="c", subcore_axis_name="s")

    @functools.partial(
        pl.kernel, mesh=mesh,
        out_type=[jax.ShapeDtypeStruct((s_rows, D), jnp.float32),
                  jax.ShapeDtypeStruct((s_rows, 128), jnp.float32)],
        scratch_types=[
            pltpu.VMEM((TPW, D), jnp.float32),
            pltpu.VMEM((TPW, 128), jnp.float32),
            pltpu.VMEM((TPW, 128), jnp.float32),
            pltpu.VMEM((TPW,), jnp.int32),
            pltpu.VMEM((TPW,), jnp.int32),
            pltpu.SemaphoreType.DMA,
        ])
    def disp(x_hbm, d0_hbm, d1_hbm, w0_hbm, w1_hbm, xs_hbm, wt_hbm,
             x_v, w0_v, w1_v, i0_v, i1_v, sem):
        wid = lax.axis_index("s") * 2 + lax.axis_index("c")
        base = wid * TPW
        pltpu.sync_copy(d0_hbm.at[pl.ds(base, TPW)], i0_v)
        pltpu.sync_copy(d1_hbm.at[pl.ds(base, TPW)], i1_v)
        pltpu.sync_copy(x_hbm.at[pl.ds(base, TPW)], x_v)
        pltpu.sync_copy(w0_hbm.at[pl.ds(base, TPW)], w0_v)
        pltpu.sync_copy(w1_hbm.at[pl.ds(base, TPW)], w1_v)
        c0 = pltpu.async_copy(x_v, xs_hbm.at[i0_v], sem)
        c1 = pltpu.async_copy(x_v, xs_hbm.at[i1_v], sem)
        c2 = pltpu.async_copy(w0_v, wt_hbm.at[i0_v], sem)
        c3 = pltpu.async_copy(w1_v, wt_hbm.at[i1_v], sem)
        c0.wait()
        c1.wait()
        c2.wait()
        c3.wait()

    return disp(x, d0, d1, w0, w1)


def _ridx(s):
    return (s // 5) * 3 + jnp.minimum(s % 5, 2)


def _sidx(s):
    return (s // 5) * 2 + jnp.clip(s % 5 - 3, 0, 1)


def _gffn_body(be_sref, xs_ref, wg_ref, wu_ref, wd_ref, wt_ref,
               x_ref, swg_ref, swu_ref, swd_ref, sg_ref,
               ys_ref, sh0_ref, sh1_ref,
               wgb_ref, wub_ref, wdb_ref, swgb_ref, swub_ref, swdb_ref):
    s = pl.program_id(0)
    p = s % 5
    is_routed = p < 3
    ridx = _ridx(s)
    cur = be_sref[ridx]
    prev = be_sref[jnp.maximum(ridx - 1, 0)]

    @pl.when(is_routed & ((ridx == 0) | (cur != prev)))
    def _():
        wgb_ref[...] = wg_ref[0].astype(jnp.bfloat16)
        wub_ref[...] = wu_ref[0].astype(jnp.bfloat16)
        wdb_ref[...] = wd_ref[0].astype(jnp.bfloat16)

    @pl.when(is_routed)
    def _():
        xb = xs_ref[...].astype(jnp.bfloat16)
        g = _dot_nt(xb, wgb_ref[...])
        u = _dot_nt(xb, wub_ref[...])
        h = ((g * _silu(u)) * wt_ref[:, :1]).astype(jnp.bfloat16)
        ys_ref[...] = _dot_nt(h, wdb_ref[...])

    @pl.when(~is_routed)
    def _():
        sidx = (s // 5) * 2 + (p - 3)
        pseudo = sidx // 8

        @pl.when(sidx % 8 == 0)
        def _():
            swgb_ref[...] = swg_ref[...].astype(jnp.bfloat16)
            swub_ref[...] = swu_ref[...].astype(jnp.bfloat16)
            swdb_ref[...] = swd_ref[...].astype(jnp.bfloat16)

        xs = x_ref[...]
        xb = xs.astype(jnp.bfloat16)
        g = _dot_nt(xb, swgb_ref[...])
        u = _dot_nt(xb, swub_ref[...])
        h = (g * _silu(u)).astype(jnp.bfloat16)
        se = _dot_nt(h, swdb_ref[...])
        gate = _sigmoid(_dot_nt(xs, sg_ref[...]))
        sh = (se * gate).astype(jnp.bfloat16)

        @pl.when(pseudo == 0)
        def _():
            sh0_ref[...] = sh

        @pl.when(pseudo == 1)
        def _():
            sh1_ref[...] = sh


def _gffn(be, xs, Wg, Wu, Wd, wt2, x, sWg, sWu, sWd, s_gate):
    S_rows, D = xs.shape
    T = x.shape[0]
    E, F, _ = Wg.shape
    nb = S_rows // _M
    n_tb = T // _M
    grid = nb + 2 * n_tb
    grid_spec = pltpu.PrefetchScalarGridSpec(
        num_scalar_prefetch=1,
        grid=(grid,),
        in_specs=[
            pl.BlockSpec((_M, D), lambda s, be_ref: (_ridx(s), 0)),
            pl.BlockSpec((1, F, D), lambda s, be_ref: (be_ref[_ridx(s)], 0, 0)),
            pl.BlockSpec((1, F, D), lambda s, be_ref: (be_ref[_ridx(s)], 0, 0)),
            pl.BlockSpec((1, D, F), lambda s, be_ref: (be_ref[_ridx(s)], 0, 0)),
            pl.BlockSpec((_M, 128), lambda s, be_ref: (_ridx(s), 0)),
            pl.BlockSpec((_M, D), lambda s, be_ref: (_sidx(s) % 8, 0)),
            pl.BlockSpec((F, D), lambda s, be_ref: (_sidx(s) // 8, 0)),
            pl.BlockSpec((F, D), lambda s, be_ref: (_sidx(s) // 8, 0)),
            pl.BlockSpec((D, F), lambda s, be_ref: (0, _sidx(s) // 8)),
            pl.BlockSpec((1, D), lambda s, be_ref: (0, 0)),
        ],
        out_specs=[
            pl.BlockSpec((_M, D), lambda s, be_ref: (_ridx(s), 0)),
            pl.BlockSpec((_M, D),
                         lambda s, be_ref: (jnp.clip(_sidx(s), 0, 7), 0)),
            pl.BlockSpec((_M, D),
                         lambda s, be_ref: (jnp.clip(_sidx(s) - 8, 0, 7), 0)),
        ],
        scratch_shapes=[pltpu.VMEM((F, D), jnp.bfloat16),
                        pltpu.VMEM((F, D), jnp.bfloat16),
                        pltpu.VMEM((D, F), jnp.bfloat16),
                        pltpu.VMEM((F, D), jnp.bfloat16),
                        pltpu.VMEM((F, D), jnp.bfloat16),
                        pltpu.VMEM((D, F), jnp.bfloat16)],
    )
    return pl.pallas_call(
        _gffn_body,
        grid_spec=grid_spec,
        out_shape=[jax.ShapeDtypeStruct((S_rows, D), jnp.float32),
                   jax.ShapeDtypeStruct((T, D), jnp.bfloat16),
                   jax.ShapeDtypeStruct((T, D), jnp.bfloat16)],
    )(be, xs, Wg, Wu, Wd, wt2, x, sWg, sWu, sWd, s_gate)


def _sc_gather(ys, d0, d1):
    S, D = ys.shape
    T = d0.shape[0]
    NW = 32
    TPW = T // NW
    mesh = plsc.VectorSubcoreMesh(core_axis_name="c", subcore_axis_name="s")

    @functools.partial(
        pl.kernel, mesh=mesh,
        out_type=[jax.ShapeDtypeStruct((T, D), jnp.float32),
                  jax.ShapeDtypeStruct((T, D), jnp.float32)],
        scratch_types=[
            pltpu.VMEM((TPW, D), jnp.float32),
            pltpu.VMEM((TPW, D), jnp.float32),
            pltpu.VMEM((TPW,), jnp.int32),
            pltpu.VMEM((TPW,), jnp.int32),
            pltpu.SemaphoreType.DMA,
        ])
    def gath(ys_hbm, d0_hbm, d1_hbm, y0_hbm, y1_hbm, y0_v, y1_v,
             i0_v, i1_v, sem):
        wid = lax.axis_index("s") * 2 + lax.axis_index("c")
        base = wid * TPW
        pltpu.sync_copy(d0_hbm.at[pl.ds(base, TPW)], i0_v)
        pltpu.sync_copy(d1_hbm.at[pl.ds(base, TPW)], i1_v)
        c0 = pltpu.async_copy(ys_hbm.at[i0_v], y0_v, sem)
        c1 = pltpu.async_copy(ys_hbm.at[i1_v], y1_v, sem)
        c0.wait()
        c1.wait()
        pltpu.sync_copy(y0_v, y0_hbm.at[pl.ds(base, TPW)])
        pltpu.sync_copy(y1_v, y1_hbm.at[pl.ds(base, TPW)])

    return gath(ys, d0, d1)


def _comb_body(sh0_ref, sh1_ref, y0_ref, y1_ref, out_ref):
    out_ref[...] = (sh0_ref[...].astype(jnp.float32)
                    + sh1_ref[...].astype(jnp.float32)
                    + y0_ref[...] + y1_ref[...])


def _combine(sh0, sh1, y0, y1):
    T, D = y0.shape
    TB = min(1024, T)
    ntb = T // TB
    return pl.pallas_call(
        _comb_body,
        grid=(ntb,),
        in_specs=[
            pl.BlockSpec((TB, D), lambda tb: (tb, 0)),
            pl.BlockSpec((TB, D), lambda tb: (tb, 0)),
            pl.BlockSpec((TB, D), lambda tb: (tb, 0)),
            pl.BlockSpec((TB, D), lambda tb: (tb, 0)),
        ],
        out_specs=pl.BlockSpec((TB, D), lambda tb: (tb, 0)),
        out_shape=jax.ShapeDtypeStruct((T, D), jnp.float32),
    )(sh0, sh1, y0, y1)


def kernel(hidden_states, gate_w, Wg, Wu, Wd, sWg, sWu, sWd, s_gate):
    x = hidden_states
    T, D = x.shape
    E = gate_w.shape[0]
    s_rows = 2 * T + E * _M
    nb = s_rows // _M
    d0, d1, w0, w1, be = _router(x, gate_w, _M, nb)
    d0f, d1f = d0.reshape(T), d1.reshape(T)
    xs, wt = _sc_dispatch(x, d0f, d1f, w0, w1, s_rows)
    ys, sh0, sh1 = _gffn(be.reshape(nb), xs, Wg, Wu, Wd, wt,
                         x, sWg, sWu, sWd, s_gate)
    y0, y1 = _sc_gather(ys, d0f, d1f)
    return _combine(sh0, sh1, y0, y1)


# speedup vs baseline: 1.0932x; 1.0932x over previous
import functools

import jax
import jax.numpy as jnp
from jax import lax
from jax.experimental import pallas as pl
from jax.experimental.pallas import tpu as pltpu
from jax.experimental.pallas import tpu_sc as plsc

_M = 256


def _silu(u):
    return u / (1.0 + jnp.exp(-u))


def _sigmoid(u):
    return 1.0 / (1.0 + jnp.exp(-u))


def _dot_nt(a, b):
    return lax.dot_general(a, b, (((1,), (1,)), ((), ())),
                           preferred_element_type=jnp.float32)


def _dot_nn(a, b):
    return lax.dot_general(a, b, (((1,), (0,)), ((), ())),
                           preferred_element_type=jnp.float32)


def _router_body(x_ref, gw_ref, d0_ref, d1_ref, w0_ref, w1_ref, be_ref,
                 *, m_blk, nb):
    logits = _dot_nt(x_ref[...], gw_ref[...])
    T, E = logits.shape
    lane = lax.broadcasted_iota(jnp.int32, (T, E), 1)
    m1 = jnp.max(logits, axis=1, keepdims=True)
    i1 = jnp.min(jnp.where(logits == m1, lane, E), axis=1, keepdims=True)
    masked = jnp.where(lane == i1, -jnp.inf, logits)
    m2 = jnp.max(masked, axis=1, keepdims=True)
    i2 = jnp.min(jnp.where(masked == m2, lane, E), axis=1, keepdims=True)
    WL = w0_ref.shape[1]
    wa = 1.0 / (1.0 + jnp.exp(m2 - m1))
    zeros_wl = jnp.zeros((T, WL), jnp.float32)
    w0_ref[...] = wa + zeros_wl
    w1_ref[...] = (1.0 - wa) + zeros_wl

    O1 = (lane == i1).astype(jnp.float32)
    O2 = (lane == i2).astype(jnp.float32)

    C = 512
    r_io = lax.broadcasted_iota(jnp.int32, (C, C), 0)
    c_io = lax.broadcasted_iota(jnp.int32, (C, C), 1)
    LT = (r_io >= c_io).astype(jnp.float32)
    n_chunk_half = T // C
    counts = jnp.zeros((1, E), jnp.float32)
    ranks = []
    for c in range(2 * n_chunk_half):
        if c < n_chunk_half:
            Oc = O1[c * C:(c + 1) * C, :]
        else:
            Oc = O2[(c - n_chunk_half) * C:(c - n_chunk_half + 1) * C, :]
        pref = _dot_nn(LT, Oc)
        ranks.append(jnp.sum((pref + counts) * Oc, axis=1, keepdims=True)
                     - 1.0)
        counts = counts + jnp.sum(Oc, axis=0, keepdims=True)

    cnt_i = counts.astype(jnp.int32)
    pc = ((cnt_i + m_blk - 1) // m_blk) * m_blk
    pc_f = pc.astype(jnp.float32)
    re_io = lax.broadcasted_iota(jnp.int32, (E, E), 0)
    ce_io = lax.broadcasted_iota(jnp.int32, (E, E), 1)
    U = (re_io < ce_io).astype(jnp.float32)
    off = _dot_nn(pc_f, U)

    off1 = jnp.sum(off * O1, axis=1, keepdims=True)
    off2 = jnp.sum(off * O2, axis=1, keepdims=True)
    rank = jnp.concatenate(ranks, axis=0)
    d0_ref[...] = (off1 + rank[:T, :]).astype(jnp.int32)
    d1_ref[...] = (off2 + rank[T:, :]).astype(jnp.int32)

    off_end = off + pc_f
    eyeE = (re_io == ce_io).astype(jnp.float32)
    oe_col = _dot_nt(eyeE, off_end)
    brow = (lax.broadcasted_iota(jnp.int32, (1, nb), 1) * m_blk
            ).astype(jnp.float32)
    cmp = (brow >= oe_col).astype(jnp.int32)
    be_ref[...] = jnp.minimum(jnp.sum(cmp, axis=0, keepdims=True), E - 1)


def _router(x, gate_w, m_blk, nb):
    T, D = x.shape
    E = gate_w.shape[0]
    body = functools.partial(_router_body, m_blk=m_blk, nb=nb)
    return pl.pallas_call(
        body,
        out_shape=[
            jax.ShapeDtypeStruct((T, 1), jnp.int32),
            jax.ShapeDtypeStruct((T, 1), jnp.int32),
            jax.ShapeDtypeStruct((T, 128), jnp.float32),
            jax.ShapeDtypeStruct((T, 128), jnp.float32),
            jax.ShapeDtypeStruct((1, nb), jnp.int32),
        ],
    )(x, gate_w)


def _sc_dispatch(x, d0, d1, w0, w1, s_rows):
    T, D = x.shape
    NW = 32
    TPW = T // NW
    mesh = plsc.VectorSubcoreMesh(core_axis_name="c", subcore_axis_name="s")

    @functools.partial(
        pl.kernel, mesh=mesh,
        out_type=[jax.ShapeDtypeStruct((s_rows, D), jnp.float32),
                  jax.ShapeDtypeStruct((s_rows, 128), jnp.float32)],
        scratch_types=[
            pltpu.VMEM((TPW, D), jnp.float32),
            pltpu.VMEM((TPW, 128), jnp.float32),
            pltpu.VMEM((TPW, 128), jnp.float32),
            pltpu.VMEM((TPW,), jnp.int32),
            pltpu.VMEM((TPW,), jnp.int32),
            pltpu.SemaphoreType.DMA,
        ])
    def disp(x_hbm, d0_hbm, d1_hbm, w0_hbm, w1_hbm, xs_hbm, wt_hbm,
             x_v, w0_v, w1_v, i0_v, i1_v, sem):
        wid = lax.axis_index("s") * 2 + lax.axis_index("c")
        base = wid * TPW
        pltpu.sync_copy(d0_hbm.at[pl.ds(base, TPW)], i0_v)
        pltpu.sync_copy(d1_hbm.at[pl.ds(base, TPW)], i1_v)
        pltpu.sync_copy(x_hbm.at[pl.ds(base, TPW)], x_v)
        pltpu.sync_copy(w0_hbm.at[pl.ds(base, TPW)], w0_v)
        pltpu.sync_copy(w1_hbm.at[pl.ds(base, TPW)], w1_v)
        c0 = pltpu.async_copy(x_v, xs_hbm.at[i0_v], sem)
        c1 = pltpu.async_copy(x_v, xs_hbm.at[i1_v], sem)
        c2 = pltpu.async_copy(w0_v, wt_hbm.at[i0_v], sem)
        c3 = pltpu.async_copy(w1_v, wt_hbm.at[i1_v], sem)
        c0.wait()
        c1.wait()
        c2.wait()
        c3.wait()

    return disp(x, d0, d1, w0, w1)


def _gffn_body(be_sref, xs_ref, wg_ref, wu_ref, wd_ref, wt_ref, ys_ref,
               wgb_ref, wub_ref, wdb_ref):
    b = pl.program_id(0)
    cur = be_sref[b]
    prev = be_sref[jnp.maximum(b - 1, 0)]

    @pl.when((b == 0) | (cur != prev))
    def _():
        wgb_ref[...] = wg_ref[0].astype(jnp.bfloat16)
        wub_ref[...] = wu_ref[0].astype(jnp.bfloat16)
        wdb_ref[...] = wd_ref[0].astype(jnp.bfloat16)

    xb = xs_ref[...].astype(jnp.bfloat16)
    g = _dot_nt(xb, wgb_ref[...])
    u = _dot_nt(xb, wub_ref[...])
    h = ((g * _silu(u)) * wt_ref[:, :1]).astype(jnp.bfloat16)
    ys_ref[...] = _dot_nt(h, wdb_ref[...])


def _gffn(be, xs, Wg, Wu, Wd, wt2):
    S, D = xs.shape
    E, F, _ = Wg.shape
    nb = S // _M
    grid_spec = pltpu.PrefetchScalarGridSpec(
        num_scalar_prefetch=1,
        grid=(nb,),
        in_specs=[
            pl.BlockSpec((_M, D), lambda b, be_ref: (b, 0)),
            pl.BlockSpec((1, F, D), lambda b, be_ref: (be_ref[b], 0, 0)),
            pl.BlockSpec((1, F, D), lambda b, be_ref: (be_ref[b], 0, 0)),
            pl.BlockSpec((1, D, F), lambda b, be_ref: (be_ref[b], 0, 0)),
            pl.BlockSpec((_M, 128), lambda b, be_ref: (b, 0)),
        ],
        out_specs=pl.BlockSpec((_M, D), lambda b, be_ref: (b, 0)),
        scratch_shapes=[pltpu.VMEM((F, D), jnp.bfloat16),
                        pltpu.VMEM((F, D), jnp.bfloat16),
                        pltpu.VMEM((D, F), jnp.bfloat16)],
    )
    return pl.pallas_call(
        _gffn_body,
        grid_spec=grid_spec,
        out_shape=jax.ShapeDtypeStruct((S, D), jnp.float32),
    )(be, xs, Wg, Wu, Wd, wt2)


def _sc_gather(ys, d0, d1):
    S, D = ys.shape
    T = d0.shape[0]
    NW = 32
    TPW = T // NW
    mesh = plsc.VectorSubcoreMesh(core_axis_name="c", subcore_axis_name="s")

    @functools.partial(
        pl.kernel, mesh=mesh,
        out_type=[jax.ShapeDtypeStruct((T, D), jnp.float32),
                  jax.ShapeDtypeStruct((T, D), jnp.float32)],
        scratch_types=[
            pltpu.VMEM((TPW, D), jnp.float32),
            pltpu.VMEM((TPW, D), jnp.float32),
            pltpu.VMEM((TPW,), jnp.int32),
            pltpu.VMEM((TPW,), jnp.int32),
            pltpu.SemaphoreType.DMA,
        ])
    def gath(ys_hbm, d0_hbm, d1_hbm, y0_hbm, y1_hbm, y0_v, y1_v,
             i0_v, i1_v, sem):
        wid = lax.axis_index("s") * 2 + lax.axis_index("c")
        base = wid * TPW
        pltpu.sync_copy(d0_hbm.at[pl.ds(base, TPW)], i0_v)
        pltpu.sync_copy(d1_hbm.at[pl.ds(base, TPW)], i1_v)
        c0 = pltpu.async_copy(ys_hbm.at[i0_v], y0_v, sem)
        c1 = pltpu.async_copy(ys_hbm.at[i1_v], y1_v, sem)
        c0.wait()
        c1.wait()
        pltpu.sync_copy(y0_v, y0_hbm.at[pl.ds(base, TPW)])
        pltpu.sync_copy(y1_v, y1_hbm.at[pl.ds(base, TPW)])

    return gath(ys, d0, d1)


def _sh_body(x_ref, swg_ref, swu_ref, swd_ref, sg_ref, out_ref,
             swgb_ref, swub_ref, swdb_ref):
    tb = pl.program_id(0)

    @pl.when(tb == 0)
    def _():
        swgb_ref[...] = swg_ref[...].astype(jnp.bfloat16)
        swub_ref[...] = swu_ref[...].astype(jnp.bfloat16)
        swdb_ref[...] = swd_ref[...].astype(jnp.bfloat16)

    xs = x_ref[...]
    xb = xs.astype(jnp.bfloat16)
    g = _dot_nt(xb, swgb_ref[...])
    u = _dot_nt(xb, swub_ref[...])
    h = (g * _silu(u)).astype(jnp.bfloat16)
    se = _dot_nt(h, swdb_ref[...])
    gate = _sigmoid(_dot_nt(xs, sg_ref[...]))
    out_ref[...] = se * gate


def _shared(x, sWg, sWu, sWd, s_gate):
    T, D = x.shape
    S = sWg.shape[0]
    TB = min(512, T)
    ntb = T // TB
    return pl.pallas_call(
        _sh_body,
        grid=(ntb,),
        in_specs=[
            pl.BlockSpec((TB, D), lambda tb: (tb, 0)),
            pl.BlockSpec((S, D), lambda tb: (0, 0)),
            pl.BlockSpec((S, D), lambda tb: (0, 0)),
            pl.BlockSpec((D, S), lambda tb: (0, 0)),
            pl.BlockSpec((1, D), lambda tb: (0, 0)),
        ],
        out_specs=pl.BlockSpec((TB, D), lambda tb: (tb, 0)),
        out_shape=jax.ShapeDtypeStruct((T, D), jnp.float32),
        scratch_shapes=[pltpu.VMEM((S, D), jnp.bfloat16),
                        pltpu.VMEM((S, D), jnp.bfloat16),
                        pltpu.VMEM((D, S), jnp.bfloat16)],
    )(x, sWg, sWu, sWd, s_gate)


def _comb_body(sh_ref, y0_ref, y1_ref, out_ref):
    out_ref[...] = sh_ref[...] + y0_ref[...] + y1_ref[...]


def _combine(sh, y0, y1):
    T, D = y0.shape
    TB = min(1024, T)
    ntb = T // TB
    return pl.pallas_call(
        _comb_body,
        grid=(ntb,),
        in_specs=[
            pl.BlockSpec((TB, D), lambda tb: (tb, 0)),
            pl.BlockSpec((TB, D), lambda tb: (tb, 0)),
            pl.BlockSpec((TB, D), lambda tb: (tb, 0)),
        ],
        out_specs=pl.BlockSpec((TB, D), lambda tb: (tb, 0)),
        out_shape=jax.ShapeDtypeStruct((T, D), jnp.float32),
    )(sh, y0, y1)


def kernel(hidden_states, gate_w, Wg, Wu, Wd, sWg, sWu, sWd, s_gate):
    x = hidden_states
    T, D = x.shape
    E = gate_w.shape[0]
    s_rows = 2 * T + E * _M
    nb = s_rows // _M
    d0, d1, w0, w1, be = _router(x, gate_w, _M, nb)
    d0f, d1f = d0.reshape(T), d1.reshape(T)
    xs, wt = _sc_dispatch(x, d0f, d1f, w0, w1, s_rows)
    sh = _shared(x, sWg, sWu, sWd, s_gate)
    ys = _gffn(be.reshape(nb), xs, Wg, Wu, Wd, wt)
    y0, y1 = _sc_gather(ys, d0f, d1f)
    return _combine(sh, y0, y1)


# speedup vs baseline: 1.1203x; 1.0248x over previous
import functools

import jax
import jax.numpy as jnp
from jax import lax
from jax.experimental import pallas as pl
from jax.experimental.pallas import tpu as pltpu
from jax.experimental.pallas import tpu_sc as plsc

_M = 256


def _silu(u):
    return u / (1.0 + jnp.exp(-u))


def _sigmoid(u):
    return 1.0 / (1.0 + jnp.exp(-u))


def _dot_nt(a, b):
    return lax.dot_general(a, b, (((1,), (1,)), ((), ())),
                           preferred_element_type=jnp.float32)


def _dot_nn(a, b):
    return lax.dot_general(a, b, (((1,), (0,)), ((), ())),
                           preferred_element_type=jnp.float32)


def _router_body(x_ref, gw_ref, d0_ref, d1_ref, w0_ref, w1_ref, be_ref,
                 *, m_blk, nb):
    logits = _dot_nt(x_ref[...], gw_ref[...])
    T, E = logits.shape
    lane = lax.broadcasted_iota(jnp.int32, (T, E), 1)
    m1 = jnp.max(logits, axis=1, keepdims=True)
    i1 = jnp.min(jnp.where(logits == m1, lane, E), axis=1, keepdims=True)
    masked = jnp.where(lane == i1, -jnp.inf, logits)
    m2 = jnp.max(masked, axis=1, keepdims=True)
    i2 = jnp.min(jnp.where(masked == m2, lane, E), axis=1, keepdims=True)
    WL = w0_ref.shape[1]
    wa = 1.0 / (1.0 + jnp.exp(m2 - m1))
    zeros_wl = jnp.zeros((T, WL), jnp.float32)
    w0_ref[...] = wa + zeros_wl
    w1_ref[...] = (1.0 - wa) + zeros_wl

    O1 = (lane == i1).astype(jnp.float32)
    O2 = (lane == i2).astype(jnp.float32)

    C = 512
    r_io = lax.broadcasted_iota(jnp.int32, (C, C), 0)
    c_io = lax.broadcasted_iota(jnp.int32, (C, C), 1)
    LT = (r_io >= c_io).astype(jnp.float32)
    n_chunk_half = T // C
    counts = jnp.zeros((1, E), jnp.float32)
    ranks = []
    for c in range(2 * n_chunk_half):
        if c < n_chunk_half:
            Oc = O1[c * C:(c + 1) * C, :]
        else:
            Oc = O2[(c - n_chunk_half) * C:(c - n_chunk_half + 1) * C, :]
        pref = _dot_nn(LT, Oc)
        ranks.append(jnp.sum((pref + counts) * Oc, axis=1, keepdims=True)
                     - 1.0)
        counts = counts + jnp.sum(Oc, axis=0, keepdims=True)

    cnt_i = counts.astype(jnp.int32)
    pc = ((cnt_i + m_blk - 1) // m_blk) * m_blk
    pc_f = pc.astype(jnp.float32)
    re_io = lax.broadcasted_iota(jnp.int32, (E, E), 0)
    ce_io = lax.broadcasted_iota(jnp.int32, (E, E), 1)
    U = (re_io < ce_io).astype(jnp.float32)
    off = _dot_nn(pc_f, U)

    off1 = jnp.sum(off * O1, axis=1, keepdims=True)
    off2 = jnp.sum(off * O2, axis=1, keepdims=True)
    rank = jnp.concatenate(ranks, axis=0)
    d0_ref[...] = (off1 + rank[:T, :]).astype(jnp.int32)
    d1_ref[...] = (off2 + rank[T:, :]).astype(jnp.int32)

    off_end = off + pc_f
    eyeE = (re_io == ce_io).astype(jnp.float32)
    oe_col = _dot_nt(eyeE, off_end)
    brow = (lax.broadcasted_iota(jnp.int32, (1, nb), 1) * m_blk
            ).astype(jnp.float32)
    cmp = (brow >= oe_col).astype(jnp.int32)
    be_ref[...] = jnp.minimum(jnp.sum(cmp, axis=0, keepdims=True), E - 1)


def _router(x, gate_w, m_blk, nb):
    T, D = x.shape
    E = gate_w.shape[0]
    body = functools.partial(_router_body, m_blk=m_blk, nb=nb)
    return pl.pallas_call(
        body,
        out_shape=[
            jax.ShapeDtypeStruct((T, 1), jnp.int32),
            jax.ShapeDtypeStruct((T, 1), jnp.int32),
            jax.ShapeDtypeStruct((T, 128), jnp.float32),
            jax.ShapeDtypeStruct((T, 128), jnp.float32),
            jax.ShapeDtypeStruct((1, nb), jnp.int32),
        ],
    )(x, gate_w)


def _sc_dispatch(x, d0, d1, w0, w1, s_rows):
    T, D = x.shape
    NW = 32
    TPW = T // NW
    mesh = plsc.VectorSubcoreMesh(core_axis_name="c", subcore_axis_name="s")

    @functools.partial(
        pl.kernel, mesh=mesh,
        out_type=[jax.ShapeDtypeStruct((s_rows, D), jnp.float32),
                  jax.ShapeDtypeStruct((s_rows, 128), jnp.float32)],
        scratch_types=[
            pltpu.VMEM((TPW, D), jnp.float32),
            pltpu.VMEM((TPW, 128), jnp.float32),
            pltpu.VMEM((TPW, 128), jnp.float32),
            pltpu.VMEM((TPW,), jnp.int32),
            pltpu.VMEM((TPW,), jnp.int32),
            pltpu.SemaphoreType.DMA,
        ])
    def disp(x_hbm, d0_hbm, d1_hbm, w0_hbm, w1_hbm, xs_hbm, wt_hbm,
             x_v, w0_v, w1_v, i0_v, i1_v, sem):
        wid = lax.axis_index("s") * 2 + lax.axis_index("c")
        base = wid * TPW
        pltpu.sync_copy(d0_hbm.at[pl.ds(base, TPW)], i0_v)
        pltpu.sync_copy(d1_hbm.at[pl.ds(base, TPW)], i1_v)
        pltpu.sync_copy(x_hbm.at[pl.ds(base, TPW)], x_v)
        pltpu.sync_copy(w0_hbm.at[pl.ds(base, TPW)], w0_v)
        pltpu.sync_copy(w1_hbm.at[pl.ds(base, TPW)], w1_v)
        c0 = pltpu.async_copy(x_v, xs_hbm.at[i0_v], sem)
        c1 = pltpu.async_copy(x_v, xs_hbm.at[i1_v], sem)
        c2 = pltpu.async_copy(w0_v, wt_hbm.at[i0_v], sem)
        c3 = pltpu.async_copy(w1_v, wt_hbm.at[i1_v], sem)
        c0.wait()
        c1.wait()
        c2.wait()
        c3.wait()

    return disp(x, d0, d1, w0, w1)


def _gffn_body(be_sref, xs_ref, wg_ref, wu_ref, wd_ref, wt_ref, ys_ref):
    xb = xs_ref[...].astype(jnp.bfloat16)
    g = _dot_nt(xb, wg_ref[0].astype(jnp.bfloat16))
    u = _dot_nt(xb, wu_ref[0].astype(jnp.bfloat16))
    h = ((g * _silu(u)) * wt_ref[:, :1]).astype(jnp.bfloat16)
    ys_ref[...] = _dot_nt(h, wd_ref[0].astype(jnp.bfloat16))


def _gffn(be, xs, Wg, Wu, Wd, wt2):
    S, D = xs.shape
    E, F, _ = Wg.shape
    nb = S // _M
    grid_spec = pltpu.PrefetchScalarGridSpec(
        num_scalar_prefetch=1,
        grid=(nb,),
        in_specs=[
            pl.BlockSpec((_M, D), lambda b, be_ref: (b, 0)),
            pl.BlockSpec((1, F, D), lambda b, be_ref: (be_ref[b], 0, 0)),
            pl.BlockSpec((1, F, D), lambda b, be_ref: (be_ref[b], 0, 0)),
            pl.BlockSpec((1, D, F), lambda b, be_ref: (be_ref[b], 0, 0)),
            pl.BlockSpec((_M, 128), lambda b, be_ref: (b, 0)),
        ],
        out_specs=pl.BlockSpec((_M, D), lambda b, be_ref: (b, 0)),
    )
    return pl.pallas_call(
        _gffn_body,
        grid_spec=grid_spec,
        out_shape=jax.ShapeDtypeStruct((S, D), jnp.float32),
    )(be, xs, Wg, Wu, Wd, wt2)


def _sc_gather(ys, d0, d1):
    S, D = ys.shape
    T = d0.shape[0]
    NW = 32
    TPW = T // NW
    mesh = plsc.VectorSubcoreMesh(core_axis_name="c", subcore_axis_name="s")

    @functools.partial(
        pl.kernel, mesh=mesh,
        out_type=[jax.ShapeDtypeStruct((T, D), jnp.float32),
                  jax.ShapeDtypeStruct((T, D), jnp.float32)],
        scratch_types=[
            pltpu.VMEM((TPW, D), jnp.float32),
            pltpu.VMEM((TPW, D), jnp.float32),
            pltpu.VMEM((TPW,), jnp.int32),
            pltpu.VMEM((TPW,), jnp.int32),
            pltpu.SemaphoreType.DMA,
        ])
    def gath(ys_hbm, d0_hbm, d1_hbm, y0_hbm, y1_hbm, y0_v, y1_v,
             i0_v, i1_v, sem):
        wid = lax.axis_index("s") * 2 + lax.axis_index("c")
        base = wid * TPW
        pltpu.sync_copy(d0_hbm.at[pl.ds(base, TPW)], i0_v)
        pltpu.sync_copy(d1_hbm.at[pl.ds(base, TPW)], i1_v)
        c0 = pltpu.async_copy(ys_hbm.at[i0_v], y0_v, sem)
        c1 = pltpu.async_copy(ys_hbm.at[i1_v], y1_v, sem)
        c0.wait()
        c1.wait()
        pltpu.sync_copy(y0_v, y0_hbm.at[pl.ds(base, TPW)])
        pltpu.sync_copy(y1_v, y1_hbm.at[pl.ds(base, TPW)])

    return gath(ys, d0, d1)


def _sh_body(x_ref, swg_ref, swu_ref, swd_ref, sg_ref, out_ref,
             swgb_ref, swub_ref, swdb_ref):
    tb = pl.program_id(0)

    @pl.when(tb == 0)
    def _():
        swgb_ref[...] = swg_ref[...].astype(jnp.bfloat16)
        swub_ref[...] = swu_ref[...].astype(jnp.bfloat16)
        swdb_ref[...] = swd_ref[...].astype(jnp.bfloat16)

    xs = x_ref[...]
    xb = xs.astype(jnp.bfloat16)
    g = _dot_nt(xb, swgb_ref[...])
    u = _dot_nt(xb, swub_ref[...])
    h = (g * _silu(u)).astype(jnp.bfloat16)
    se = _dot_nt(h, swdb_ref[...])
    gate = _sigmoid(_dot_nt(xs, sg_ref[...]))
    out_ref[...] = se * gate


def _shared(x, sWg, sWu, sWd, s_gate):
    T, D = x.shape
    S = sWg.shape[0]
    TB = min(512, T)
    ntb = T // TB
    return pl.pallas_call(
        _sh_body,
        grid=(ntb,),
        in_specs=[
            pl.BlockSpec((TB, D), lambda tb: (tb, 0)),
            pl.BlockSpec((S, D), lambda tb: (0, 0)),
            pl.BlockSpec((S, D), lambda tb: (0, 0)),
            pl.BlockSpec((D, S), lambda tb: (0, 0)),
            pl.BlockSpec((1, D), lambda tb: (0, 0)),
        ],
        out_specs=pl.BlockSpec((TB, D), lambda tb: (tb, 0)),
        out_shape=jax.ShapeDtypeStruct((T, D), jnp.float32),
        scratch_shapes=[pltpu.VMEM((S, D), jnp.bfloat16),
                        pltpu.VMEM((S, D), jnp.bfloat16),
                        pltpu.VMEM((D, S), jnp.bfloat16)],
    )(x, sWg, sWu, sWd, s_gate)


def _comb_body(sh_ref, y0_ref, y1_ref, out_ref):
    out_ref[...] = sh_ref[...] + y0_ref[...] + y1_ref[...]


def _combine(sh, y0, y1):
    T, D = y0.shape
    TB = min(1024, T)
    ntb = T // TB
    return pl.pallas_call(
        _comb_body,
        grid=(ntb,),
        in_specs=[
            pl.BlockSpec((TB, D), lambda tb: (tb, 0)),
            pl.BlockSpec((TB, D), lambda tb: (tb, 0)),
            pl.BlockSpec((TB, D), lambda tb: (tb, 0)),
        ],
        out_specs=pl.BlockSpec((TB, D), lambda tb: (tb, 0)),
        out_shape=jax.ShapeDtypeStruct((T, D), jnp.float32),
    )(sh, y0, y1)


def kernel(hidden_states, gate_w, Wg, Wu, Wd, sWg, sWu, sWd, s_gate):
    x = hidden_states
    T, D = x.shape
    E = gate_w.shape[0]
    s_rows = 2 * T + E * _M
    nb = s_rows // _M
    d0, d1, w0, w1, be = _router(x, gate_w, _M, nb)
    d0f, d1f = d0.reshape(T), d1.reshape(T)
    xs, wt = _sc_dispatch(x, d0f, d1f, w0, w1, s_rows)
    sh = _shared(x, sWg, sWu, sWd, s_gate)
    ys = _gffn(be.reshape(nb), xs, Wg, Wu, Wd, wt)
    y0, y1 = _sc_gather(ys, d0f, d1f)
    return _combine(sh, y0, y1)


# speedup vs baseline: 1.1380x; 1.0158x over previous
import functools

import jax
import jax.numpy as jnp
from jax import lax
from jax.experimental import pallas as pl
from jax.experimental.pallas import tpu as pltpu
from jax.experimental.pallas import tpu_sc as plsc

_M = 512


def _silu(u):
    return u / (1.0 + jnp.exp(-u))


def _sigmoid(u):
    return 1.0 / (1.0 + jnp.exp(-u))


def _dot_nt(a, b):
    return lax.dot_general(a, b, (((1,), (1,)), ((), ())),
                           preferred_element_type=jnp.float32)


def _dot_nn(a, b):
    return lax.dot_general(a, b, (((1,), (0,)), ((), ())),
                           preferred_element_type=jnp.float32)


def _router_body(x_ref, gw_ref, d0_ref, d1_ref, w0_ref, w1_ref, be_ref,
                 *, m_blk, nb):
    logits = _dot_nt(x_ref[...], gw_ref[...])
    T, E = logits.shape
    lane = lax.broadcasted_iota(jnp.int32, (T, E), 1)
    m1 = jnp.max(logits, axis=1, keepdims=True)
    i1 = jnp.min(jnp.where(logits == m1, lane, E), axis=1, keepdims=True)
    masked = jnp.where(lane == i1, -jnp.inf, logits)
    m2 = jnp.max(masked, axis=1, keepdims=True)
    i2 = jnp.min(jnp.where(masked == m2, lane, E), axis=1, keepdims=True)
    WL = w0_ref.shape[1]
    wa = 1.0 / (1.0 + jnp.exp(m2 - m1))
    zeros_wl = jnp.zeros((T, WL), jnp.float32)
    w0_ref[...] = wa + zeros_wl
    w1_ref[...] = (1.0 - wa) + zeros_wl

    O1 = (lane == i1).astype(jnp.float32)
    O2 = (lane == i2).astype(jnp.float32)

    C = 512
    r_io = lax.broadcasted_iota(jnp.int32, (C, C), 0)
    c_io = lax.broadcasted_iota(jnp.int32, (C, C), 1)
    LT = (r_io >= c_io).astype(jnp.float32)
    n_chunk_half = T // C
    counts = jnp.zeros((1, E), jnp.float32)
    ranks = []
    for c in range(2 * n_chunk_half):
        if c < n_chunk_half:
            Oc = O1[c * C:(c + 1) * C, :]
        else:
            Oc = O2[(c - n_chunk_half) * C:(c - n_chunk_half + 1) * C, :]
        pref = _dot_nn(LT, Oc)
        ranks.append(jnp.sum((pref + counts) * Oc, axis=1, keepdims=True)
                     - 1.0)
        counts = counts + jnp.sum(Oc, axis=0, keepdims=True)

    cnt_i = counts.astype(jnp.int32)
    pc = ((cnt_i + m_blk - 1) // m_blk) * m_blk
    pc_f = pc.astype(jnp.float32)
    re_io = lax.broadcasted_iota(jnp.int32, (E, E), 0)
    ce_io = lax.broadcasted_iota(jnp.int32, (E, E), 1)
    U = (re_io < ce_io).astype(jnp.float32)
    off = _dot_nn(pc_f, U)

    off1 = jnp.sum(off * O1, axis=1, keepdims=True)
    off2 = jnp.sum(off * O2, axis=1, keepdims=True)
    rank = jnp.concatenate(ranks, axis=0)
    d0_ref[...] = (off1 + rank[:T, :]).astype(jnp.int32)
    d1_ref[...] = (off2 + rank[T:, :]).astype(jnp.int32)

    off_end = off + pc_f
    eyeE = (re_io == ce_io).astype(jnp.float32)
    oe_col = _dot_nt(eyeE, off_end)
    brow = (lax.broadcasted_iota(jnp.int32, (1, nb), 1) * m_blk
            ).astype(jnp.float32)
    cmp = (brow >= oe_col).astype(jnp.int32)
    be_ref[...] = jnp.minimum(jnp.sum(cmp, axis=0, keepdims=True), E - 1)


def _router(x, gate_w, m_blk, nb):
    T, D = x.shape
    E = gate_w.shape[0]
    body = functools.partial(_router_body, m_blk=m_blk, nb=nb)
    return pl.pallas_call(
        body,
        out_shape=[
            jax.ShapeDtypeStruct((T, 1), jnp.int32),
            jax.ShapeDtypeStruct((T, 1), jnp.int32),
            jax.ShapeDtypeStruct((T, 128), jnp.float32),
            jax.ShapeDtypeStruct((T, 128), jnp.float32),
            jax.ShapeDtypeStruct((1, nb), jnp.int32),
        ],
    )(x, gate_w)


def _sc_dispatch(x, d0, d1, w0, w1, s_rows):
    T, D = x.shape
    NW = 32
    TPW = T // NW
    mesh = plsc.VectorSubcoreMesh(core_axis_name="c", subcore_axis_name="s")

    @functools.partial(
        pl.kernel, mesh=mesh,
        out_type=[jax.ShapeDtypeStruct((s_rows, D), jnp.float32),
                  jax.ShapeDtypeStruct((s_rows, 128), jnp.float32)],
        scratch_types=[
            pltpu.VMEM((TPW, D), jnp.float32),
            pltpu.VMEM((TPW, 128), jnp.float32),
            pltpu.VMEM((TPW, 128), jnp.float32),
            pltpu.VMEM((TPW,), jnp.int32),
            pltpu.VMEM((TPW,), jnp.int32),
            pltpu.SemaphoreType.DMA,
        ])
    def disp(x_hbm, d0_hbm, d1_hbm, w0_hbm, w1_hbm, xs_hbm, wt_hbm,
             x_v, w0_v, w1_v, i0_v, i1_v, sem):
        wid = lax.axis_index("s") * 2 + lax.axis_index("c")
        base = wid * TPW
        pltpu.sync_copy(d0_hbm.at[pl.ds(base, TPW)], i0_v)
        pltpu.sync_copy(d1_hbm.at[pl.ds(base, TPW)], i1_v)
        pltpu.sync_copy(x_hbm.at[pl.ds(base, TPW)], x_v)
        pltpu.sync_copy(w0_hbm.at[pl.ds(base, TPW)], w0_v)
        pltpu.sync_copy(w1_hbm.at[pl.ds(base, TPW)], w1_v)
        c0 = pltpu.async_copy(x_v, xs_hbm.at[i0_v], sem)
        c1 = pltpu.async_copy(x_v, xs_hbm.at[i1_v], sem)
        c2 = pltpu.async_copy(w0_v, wt_hbm.at[i0_v], sem)
        c3 = pltpu.async_copy(w1_v, wt_hbm.at[i1_v], sem)
        c0.wait()
        c1.wait()
        c2.wait()
        c3.wait()

    return disp(x, d0, d1, w0, w1)


def _gffn_body(be_sref, xs_ref, wg_ref, wu_ref, wd_ref, wt_ref, ys_ref):
    xb = xs_ref[...].astype(jnp.bfloat16)
    g = _dot_nt(xb, wg_ref[0].astype(jnp.bfloat16))
    u = _dot_nt(xb, wu_ref[0].astype(jnp.bfloat16))
    h = ((g * _silu(u)) * wt_ref[:, :1]).astype(jnp.bfloat16)
    ys_ref[...] = _dot_nt(h, wd_ref[0].astype(jnp.bfloat16))


def _gffn(be, xs, Wg, Wu, Wd, wt2):
    S, D = xs.shape
    E, F, _ = Wg.shape
    nb = S // _M
    grid_spec = pltpu.PrefetchScalarGridSpec(
        num_scalar_prefetch=1,
        grid=(nb,),
        in_specs=[
            pl.BlockSpec((_M, D), lambda b, be_ref: (b, 0)),
            pl.BlockSpec((1, F, D), lambda b, be_ref: (be_ref[b], 0, 0)),
            pl.BlockSpec((1, F, D), lambda b, be_ref: (be_ref[b], 0, 0)),
            pl.BlockSpec((1, D, F), lambda b, be_ref: (be_ref[b], 0, 0)),
            pl.BlockSpec((_M, 128), lambda b, be_ref: (b, 0)),
        ],
        out_specs=pl.BlockSpec((_M, D), lambda b, be_ref: (b, 0)),
    )
    return pl.pallas_call(
        _gffn_body,
        grid_spec=grid_spec,
        out_shape=jax.ShapeDtypeStruct((S, D), jnp.float32),
    )(be, xs, Wg, Wu, Wd, wt2)


def _sc_gather(ys, d0, d1):
    S, D = ys.shape
    T = d0.shape[0]
    NW = 32
    TPW = T // NW
    mesh = plsc.VectorSubcoreMesh(core_axis_name="c", subcore_axis_name="s")

    @functools.partial(
        pl.kernel, mesh=mesh,
        out_type=[jax.ShapeDtypeStruct((T, D), jnp.float32),
                  jax.ShapeDtypeStruct((T, D), jnp.float32)],
        scratch_types=[
            pltpu.VMEM((TPW, D), jnp.float32),
            pltpu.VMEM((TPW, D), jnp.float32),
            pltpu.VMEM((TPW,), jnp.int32),
            pltpu.VMEM((TPW,), jnp.int32),
            pltpu.SemaphoreType.DMA,
        ])
    def gath(ys_hbm, d0_hbm, d1_hbm, y0_hbm, y1_hbm, y0_v, y1_v,
             i0_v, i1_v, sem):
        wid = lax.axis_index("s") * 2 + lax.axis_index("c")
        base = wid * TPW
        pltpu.sync_copy(d0_hbm.at[pl.ds(base, TPW)], i0_v)
        pltpu.sync_copy(d1_hbm.at[pl.ds(base, TPW)], i1_v)
        c0 = pltpu.async_copy(ys_hbm.at[i0_v], y0_v, sem)
        c1 = pltpu.async_copy(ys_hbm.at[i1_v], y1_v, sem)
        c0.wait()
        c1.wait()
        pltpu.sync_copy(y0_v, y0_hbm.at[pl.ds(base, TPW)])
        pltpu.sync_copy(y1_v, y1_hbm.at[pl.ds(base, TPW)])

    return gath(ys, d0, d1)


def _sh_body(x_ref, swg_ref, swu_ref, swd_ref, sg_ref, out_ref,
             swgb_ref, swub_ref, swdb_ref):
    tb = pl.program_id(0)

    @pl.when(tb == 0)
    def _():
        swgb_ref[...] = swg_ref[...].astype(jnp.bfloat16)
        swub_ref[...] = swu_ref[...].astype(jnp.bfloat16)
        swdb_ref[...] = swd_ref[...].astype(jnp.bfloat16)

    xs = x_ref[...]
    xb = xs.astype(jnp.bfloat16)
    g = _dot_nt(xb, swgb_ref[...])
    u = _dot_nt(xb, swub_ref[...])
    h = (g * _silu(u)).astype(jnp.bfloat16)
    se = _dot_nt(h, swdb_ref[...])
    gate = _sigmoid(_dot_nt(xs, sg_ref[...]))
    out_ref[...] = se * gate


def _shared(x, sWg, sWu, sWd, s_gate):
    T, D = x.shape
    S = sWg.shape[0]
    TB = min(512, T)
    ntb = T // TB
    return pl.pallas_call(
        _sh_body,
        grid=(ntb,),
        in_specs=[
            pl.BlockSpec((TB, D), lambda tb: (tb, 0)),
            pl.BlockSpec((S, D), lambda tb: (0, 0)),
            pl.BlockSpec((S, D), lambda tb: (0, 0)),
            pl.BlockSpec((D, S), lambda tb: (0, 0)),
            pl.BlockSpec((1, D), lambda tb: (0, 0)),
        ],
        out_specs=pl.BlockSpec((TB, D), lambda tb: (tb, 0)),
        out_shape=jax.ShapeDtypeStruct((T, D), jnp.float32),
        scratch_shapes=[pltpu.VMEM((S, D), jnp.bfloat16),
                        pltpu.VMEM((S, D), jnp.bfloat16),
                        pltpu.VMEM((D, S), jnp.bfloat16)],
    )(x, sWg, sWu, sWd, s_gate)


def _comb_body(sh_ref, y0_ref, y1_ref, out_ref):
    out_ref[...] = sh_ref[...] + y0_ref[...] + y1_ref[...]


def _combine(sh, y0, y1):
    T, D = y0.shape
    TB = min(1024, T)
    ntb = T // TB
    return pl.pallas_call(
        _comb_body,
        grid=(ntb,),
        in_specs=[
            pl.BlockSpec((TB, D), lambda tb: (tb, 0)),
            pl.BlockSpec((TB, D), lambda tb: (tb, 0)),
            pl.BlockSpec((TB, D), lambda tb: (tb, 0)),
        ],
        out_specs=pl.BlockSpec((TB, D), lambda tb: (tb, 0)),
        out_shape=jax.ShapeDtypeStruct((T, D), jnp.float32),
    )(sh, y0, y1)


def kernel(hidden_states, gate_w, Wg, Wu, Wd, sWg, sWu, sWd, s_gate):
    x = hidden_states
    T, D = x.shape
    E = gate_w.shape[0]
    s_rows = 2 * T + E * _M
    nb = s_rows // _M
    d0, d1, w0, w1, be = _router(x, gate_w, _M, nb)
    d0f, d1f = d0.reshape(T), d1.reshape(T)
    xs, wt = _sc_dispatch(x, d0f, d1f, w0, w1, s_rows)
    sh = _shared(x, sWg, sWu, sWd, s_gate)
    ys = _gffn(be.reshape(nb), xs, Wg, Wu, Wd, wt)
    y0, y1 = _sc_gather(ys, d0f, d1f)
    return _combine(sh, y0, y1)
